# baseline (speedup 1.0000x reference)
import jax
import jax.numpy as jnp
import numpy as np
from jax import lax
from jax.experimental import pallas as pl
from jax.experimental.pallas import tpu as pltpu
from jax.experimental.pallas import tpu_sc as plsc

N = 10000
E = 320000
D = 128

NC = 2
NS = 16
DH = D // NC
WH = DH // 2
CHUNK = 80
NCHUNK = 256
EW = CHUNK * NCHUNK
NBUF = 2
NLS = 4
NT = NCHUNK // NBUF
RPT = 624


def _mm_body(x_ref, w_ref, o_ref):
    o_ref[...] = jnp.dot(x_ref[...], w_ref[...],
                         preferred_element_type=jnp.float32).astype(
                             jnp.bfloat16)


def _tc_matmul(x, w):
    bm = 1000
    return pl.pallas_call(
        _mm_body,
        grid=(N // bm,),
        in_specs=[
            pl.BlockSpec((bm, D), lambda i: (i, 0)),
            pl.BlockSpec((D, D), lambda i: (0, 0)),
        ],
        out_specs=pl.BlockSpec((bm, D), lambda i: (i, 0)),
        out_shape=jax.ShapeDtypeStruct((N, D), jnp.bfloat16),
    )(x, w)


def _bcast_lane(vsl, lane):
    return lax.gather(
        vsl, jnp.full((16, 1), lane, jnp.int32),
        lax.GatherDimensionNumbers(
            offset_dims=(), collapsed_slice_dims=(0,),
            start_index_map=(0,)),
        (1,), mode=lax.GatherScatterMode.PROMISE_IN_BOUNDS)


def _sc_spmm_body(base_hbm, rowcol_hbm, val_hbm, out_hbm,
                  rc2d_v, val2d_v, gb0, gb1, fb0, fb1,
                  rowl_v, coll_v, base_sh, acc_sh,
                  g0, g1, s0, s1):
    cid = lax.axis_index("c")
    sid = lax.axis_index("s")
    gbufs = [gb0, gb1]
    fbufs = [fb0, fb1]
    gsems = [g0, g1]
    ssems = [s0, s1]
    my_base = base_hbm.at[:, pl.ds(cid * WH, WH)]
    r0 = sid * RPT

    for r in range(CHUNK):
        for jj in range(DH // 16):
            fb0[r, pl.ds(jj * 16, 16)] = jnp.zeros((16,), jnp.float32)
    for k in range(RPT // CHUNK):
        pltpu.sync_copy(fb0, acc_sh.at[pl.ds(r0 + k * CHUNK, CHUNK), :])
    pltpu.sync_copy(fb0.at[pl.ds(0, RPT % CHUNK), :],
                    acc_sh.at[pl.ds(r0 + RPT - RPT % CHUNK, RPT % CHUNK), :])

    @pl.when(sid == NS - 1)
    def _():
        pltpu.sync_copy(fb0.at[pl.ds(0, 16), :],
                        acc_sh.at[pl.ds(NS * RPT, 16), :])

    pltpu.sync_copy(my_base.at[pl.ds(r0, RPT), :],
                    base_sh.at[pl.ds(r0, RPT), :])

    @pl.when(sid == NS - 1)
    def _():
        pltpu.sync_copy(my_base.at[pl.ds(NS * RPT, 16), :],
                        base_sh.at[pl.ds(NS * RPT, 16), :])

    pltpu.sync_copy(rowcol_hbm.at[sid], rc2d_v)
    pltpu.sync_copy(val_hbm.at[sid], val2d_v)
    plsc.subcore_barrier()

    shift16 = jnp.full((16,), 16, jnp.int32)
    himask = jnp.full((16,), -65536, jnp.int32)
    lomask = jnp.full((16,), 65535, jnp.int32)

    def unpack_lists(midx, ls):
        for g in range(CHUNK // 16):
            sl = pl.ds(g * 16, 16)
            rc = rc2d_v[midx, sl]
            coll_v[ls, sl] = lax.bitwise_and(rc, lomask)
            rowl_v[ls, sl] = lax.shift_right_logical(rc, shift16)

    def scale_chunk(idx, gbuf, fbuf):
        vs = [val2d_v[idx, pl.ds(g * 16, 16)] for g in range(CHUNK // 16)]
        for g in range(CHUNK // 16):
            for lane in range(16):
                vb = _bcast_lane(vs[g], lane)
                e = g * 16 + lane
                for jj in range(DH // 32):
                    w = gbuf[e, pl.ds(jj * 16, 16)]
                    lo = lax.bitcast_convert_type(
                        lax.shift_left(w, shift16), jnp.float32)
                    hi = lax.bitcast_convert_type(
                        lax.bitwise_and(w, himask), jnp.float32)
                    fbuf[e, pl.ds(jj * 32, 16)] = lo * vb
                    fbuf[e, pl.ds(jj * 32 + 16, 16)] = hi * vb

    for p in range(NBUF):
        unpack_lists(p, p)
        pltpu.async_copy(base_sh.at[coll_v.at[p]], gbufs[p], gsems[p])

    def chunk_loop(t, carry):
        for b in range(NBUF):
            idx = NBUF * t + b
            mb = idx % NLS
            nmb = (idx + 2) % NLS
            nidx = idx + 2
            pidx = idx - 2

            pltpu.make_async_copy(
                base_sh.at[coll_v.at[mb]], gbufs[b], gsems[b]).wait()

            @pl.when(t > 0)
            def _():
                pltpu.make_async_copy(
                    fbufs[b], acc_sh.at[rowl_v.at[(pidx % NLS)]],
                    ssems[b]).wait()

            scale_chunk(idx, gbufs[b], fbufs[b])

            @pl.when(t < NT - 1)
            def _():
                unpack_lists(nidx, nmb)
                pltpu.async_copy(
                    base_sh.at[coll_v.at[nmb]], gbufs[b], gsems[b])

            pltpu.async_copy(
                fbufs[b], acc_sh.at[rowl_v.at[mb]], ssems[b], add=True)
        return carry

    lax.fori_loop(0, NT, chunk_loop, 0)
    pltpu.make_async_copy(
        fbufs[0], acc_sh.at[rowl_v.at[(NCHUNK - 2) % NLS]], ssems[0]).wait()
    pltpu.make_async_copy(
        fbufs[1], acc_sh.at[rowl_v.at[(NCHUNK - 1) % NLS]], ssems[1]).wait()

    plsc.subcore_barrier()
    pltpu.sync_copy(acc_sh.at[pl.ds(r0, RPT), :],
                    out_hbm.at[cid, pl.ds(r0, RPT), :])

    @pl.when(sid == NS - 1)
    def _():
        pltpu.sync_copy(acc_sh.at[pl.ds(NS * RPT, 16), :],
                        out_hbm.at[cid, pl.ds(NS * RPT, 16), :])


def _sc_spmm(base32, rowcol, val):
    mesh = plsc.VectorSubcoreMesh(core_axis_name="c", subcore_axis_name="s")
    f = pl.kernel(
        _sc_spmm_body,
        out_type=jax.ShapeDtypeStruct((NC, N, DH), jnp.float32),
        mesh=mesh,
        compiler_params=pltpu.CompilerParams(use_tc_tiling_on_sc=False),
        scratch_types=[
            pltpu.VMEM((NCHUNK, CHUNK), jnp.int32),
            pltpu.VMEM((NCHUNK, CHUNK), jnp.float32),
            pltpu.VMEM((CHUNK, WH), jnp.int32),
            pltpu.VMEM((CHUNK, WH), jnp.int32),
            pltpu.VMEM((CHUNK, DH), jnp.float32),
            pltpu.VMEM((CHUNK, DH), jnp.float32),
            pltpu.VMEM((NLS, CHUNK), jnp.int32),
            pltpu.VMEM((NLS, CHUNK), jnp.int32),
            pltpu.VMEM_SHARED((N, WH), jnp.int32),
            pltpu.VMEM_SHARED((N, DH), jnp.float32),
            pltpu.SemaphoreType.DMA,
            pltpu.SemaphoreType.DMA,
            pltpu.SemaphoreType.DMA,
            pltpu.SemaphoreType.DMA,
        ],
    )
    return f(base32, rowcol, val)


def _fin_body(p_ref, b_ref, g_ref, bt_ref, o_ref):
    h = jnp.concatenate([p_ref[0], p_ref[1]], axis=-1) + b_ref[...]
    h = jnp.where(h > 0, h, jnp.exp(jnp.minimum(h, 0.0)) - 1.0)
    mean = jnp.mean(h, axis=-1, keepdims=True)
    var = jnp.mean((h - mean) * (h - mean), axis=-1, keepdims=True)
    o_ref[...] = (h - mean) / jnp.sqrt(var + 1e-5) * g_ref[...] + bt_ref[...]


def _tc_finish(partials, b, gamma, beta):
    bm = 1000
    return pl.pallas_call(
        _fin_body,
        grid=(N // bm,),
        in_specs=[
            pl.BlockSpec((NC, bm, DH), lambda i: (0, i, 0)),
            pl.BlockSpec((1, D), lambda i: (0, 0)),
            pl.BlockSpec((1, D), lambda i: (0, 0)),
            pl.BlockSpec((1, D), lambda i: (0, 0)),
        ],
        out_specs=pl.BlockSpec((bm, D), lambda i: (i, 0)),
        out_shape=jax.ShapeDtypeStruct((N, D), jnp.float32),
    )(partials, b, gamma, beta)


_PERM = np.stack([np.arange(16), np.arange(16) + 16], 1).reshape(32)
_PERM = (_PERM[None, :] + 32 * np.arange(D // 32)[:, None]).reshape(D)


@jax.jit
def kernel(adj_indices, adj_values, features, W, b, gamma, beta):
    base_bf = _tc_matmul(features, W[:, _PERM])
    base32 = lax.bitcast_convert_type(
        base_bf.reshape(N, D // 2, 2), jnp.int32)
    pad = NS * EW - E
    rowcol = jnp.concatenate(
        [adj_indices[0] << 16 | adj_indices[1], jnp.zeros((pad,), jnp.int32)])
    val = jnp.concatenate([adj_values, jnp.zeros((pad,), jnp.float32)])
    partials = _sc_spmm(base32,
                        rowcol.reshape(NS, NCHUNK, CHUNK),
                        val.reshape(NS, NCHUNK, CHUNK))
    return _tc_finish(partials, b,
                      gamma.reshape(1, D), beta.reshape(1, D))


# speedup vs baseline: 1.0880x; 1.0880x over previous
import jax
import jax.numpy as jnp
import numpy as np
from jax import lax
from jax.experimental import pallas as pl
from jax.experimental.pallas import tpu as pltpu
from jax.experimental.pallas import tpu_sc as plsc

N = 10000
E = 320000
D = 128

NC = 2
NS = 16
DH = D // NC
WH = DH // 2
CHUNK = 80
NCHUNK = 256
EW = CHUNK * NCHUNK
NBUF = 2
NLS = 4
NT = NCHUNK // NBUF
RPT = 624


def _mm_body(x_ref, w_ref, o_ref):
    o_ref[...] = jnp.dot(x_ref[...], w_ref[...],
                         preferred_element_type=jnp.float32)


def _tc_matmul(x, w):
    bm = 1000
    return pl.pallas_call(
        _mm_body,
        grid=(N // bm,),
        in_specs=[
            pl.BlockSpec((bm, D), lambda i: (i, 0)),
            pl.BlockSpec((D, D), lambda i: (0, 0)),
        ],
        out_specs=pl.BlockSpec((bm, D), lambda i: (i, 0)),
        out_shape=jax.ShapeDtypeStruct((N, D), jnp.float32),
    )(x, w)


def _bcast_lane(vsl, lane):
    return lax.gather(
        vsl, jnp.full((16, 1), lane, jnp.int32),
        lax.GatherDimensionNumbers(
            offset_dims=(), collapsed_slice_dims=(0,),
            start_index_map=(0,)),
        (1,), mode=lax.GatherScatterMode.PROMISE_IN_BOUNDS)


def _sc_spmm_body(base_hbm, rowcol_hbm, val_hbm, out_hbm,
                  rc2d_v, val2d_v, gb0, gb1, fb0, fb1,
                  rowl_v, coll_v, base_sh, acc_sh,
                  g0, g1, s0, s1):
    cid = lax.axis_index("c")
    sid = lax.axis_index("s")
    gbufs = [gb0, gb1]
    fbufs = [fb0, fb1]
    gsems = [g0, g1]
    ssems = [s0, s1]
    my_base = base_hbm.at[cid]
    r0 = sid * RPT

    for r in range(CHUNK):
        for jj in range(DH // 16):
            fb0[r, pl.ds(jj * 16, 16)] = jnp.zeros((16,), jnp.float32)
    for k in range(RPT // CHUNK):
        pltpu.sync_copy(fb0, acc_sh.at[pl.ds(r0 + k * CHUNK, CHUNK), :])
    pltpu.sync_copy(fb0.at[pl.ds(0, RPT % CHUNK), :],
                    acc_sh.at[pl.ds(r0 + RPT - RPT % CHUNK, RPT % CHUNK), :])

    @pl.when(sid == NS - 1)
    def _():
        pltpu.sync_copy(fb0.at[pl.ds(0, 16), :],
                        acc_sh.at[pl.ds(NS * RPT, 16), :])

    pltpu.sync_copy(my_base.at[pl.ds(r0, RPT), :],
                    base_sh.at[pl.ds(r0, RPT), :])

    @pl.when(sid == NS - 1)
    def _():
        pltpu.sync_copy(my_base.at[pl.ds(NS * RPT, 16), :],
                        base_sh.at[pl.ds(NS * RPT, 16), :])

    pltpu.sync_copy(rowcol_hbm.at[sid], rc2d_v)
    pltpu.sync_copy(val_hbm.at[sid], val2d_v)
    plsc.subcore_barrier()

    shift16 = jnp.full((16,), 16, jnp.int32)
    himask = jnp.full((16,), -65536, jnp.int32)
    lomask = jnp.full((16,), 65535, jnp.int32)

    def unpack_lists(midx, ls):
        for g in range(CHUNK // 16):
            sl = pl.ds(g * 16, 16)
            rc = rc2d_v[midx, sl]
            coll_v[ls, sl] = lax.bitwise_and(rc, lomask)
            rowl_v[ls, sl] = lax.shift_right_logical(rc, shift16)

    def scale_chunk(idx, gbuf, fbuf):
        vs = [val2d_v[idx, pl.ds(g * 16, 16)] for g in range(CHUNK // 16)]
        for g in range(CHUNK // 16):
            for lane in range(16):
                vb = _bcast_lane(vs[g], lane)
                e = g * 16 + lane
                for jj in range(DH // 32):
                    w = gbuf[e, pl.ds(jj * 16, 16)]
                    lo = lax.bitcast_convert_type(
                        lax.shift_left(w, shift16), jnp.float32)
                    hi = lax.bitcast_convert_type(
                        lax.bitwise_and(w, himask), jnp.float32)
                    fbuf[e, pl.ds(jj * 32, 16)] = lo * vb
                    fbuf[e, pl.ds(jj * 32 + 16, 16)] = hi * vb

    for p in range(NBUF):
        unpack_lists(p, p)
        pltpu.async_copy(base_sh.at[coll_v.at[p]], gbufs[p], gsems[p])

    def chunk_loop(t, carry):
        for b in range(NBUF):
            idx = NBUF * t + b
            mb = idx % NLS
            nmb = (idx + 2) % NLS
            nidx = idx + 2
            pidx = idx - 2

            pltpu.make_async_copy(
                base_sh.at[coll_v.at[mb]], gbufs[b], gsems[b]).wait()

            @pl.when(t > 0)
            def _():
                pltpu.make_async_copy(
                    fbufs[b], acc_sh.at[rowl_v.at[(pidx % NLS)]],
                    ssems[b]).wait()

            scale_chunk(idx, gbufs[b], fbufs[b])

            @pl.when(t < NT - 1)
            def _():
                unpack_lists(nidx, nmb)
                pltpu.async_copy(
                    base_sh.at[coll_v.at[nmb]], gbufs[b], gsems[b])

            pltpu.async_copy(
                fbufs[b], acc_sh.at[rowl_v.at[mb]], ssems[b], add=True)
        return carry

    lax.fori_loop(0, NT, chunk_loop, 0)
    pltpu.make_async_copy(
        fbufs[0], acc_sh.at[rowl_v.at[(NCHUNK - 2) % NLS]], ssems[0]).wait()
    pltpu.make_async_copy(
        fbufs[1], acc_sh.at[rowl_v.at[(NCHUNK - 1) % NLS]], ssems[1]).wait()

    plsc.subcore_barrier()
    pltpu.sync_copy(acc_sh.at[pl.ds(r0, RPT), :],
                    out_hbm.at[cid, pl.ds(r0, RPT), :])

    @pl.when(sid == NS - 1)
    def _():
        pltpu.sync_copy(acc_sh.at[pl.ds(NS * RPT, 16), :],
                        out_hbm.at[cid, pl.ds(NS * RPT, 16), :])


def _sc_spmm(base32, rowcol, val):
    mesh = plsc.VectorSubcoreMesh(core_axis_name="c", subcore_axis_name="s")
    f = pl.kernel(
        _sc_spmm_body,
        out_type=jax.ShapeDtypeStruct((NC, N, DH), jnp.float32),
        mesh=mesh,
        compiler_params=pltpu.CompilerParams(use_tc_tiling_on_sc=False),
        scratch_types=[
            pltpu.VMEM((NCHUNK, CHUNK), jnp.int32),
            pltpu.VMEM((NCHUNK, CHUNK), jnp.float32),
            pltpu.VMEM((CHUNK, WH), jnp.int32),
            pltpu.VMEM((CHUNK, WH), jnp.int32),
            pltpu.VMEM((CHUNK, DH), jnp.float32),
            pltpu.VMEM((CHUNK, DH), jnp.float32),
            pltpu.VMEM((NLS, CHUNK), jnp.int32),
            pltpu.VMEM((NLS, CHUNK), jnp.int32),
            pltpu.VMEM_SHARED((N, WH), jnp.int32),
            pltpu.VMEM_SHARED((N, DH), jnp.float32),
            pltpu.SemaphoreType.DMA,
            pltpu.SemaphoreType.DMA,
            pltpu.SemaphoreType.DMA,
            pltpu.SemaphoreType.DMA,
        ],
    )
    return f(base32, rowcol, val)


def _fin_body(p_ref, b_ref, g_ref, bt_ref, o_ref):
    h = jnp.concatenate([p_ref[0], p_ref[1]], axis=-1) + b_ref[...]
    h = jnp.where(h > 0, h, jnp.exp(jnp.minimum(h, 0.0)) - 1.0)
    mean = jnp.mean(h, axis=-1, keepdims=True)
    var = jnp.mean((h - mean) * (h - mean), axis=-1, keepdims=True)
    o_ref[...] = (h - mean) / jnp.sqrt(var + 1e-5) * g_ref[...] + bt_ref[...]


def _tc_finish(partials, b, gamma, beta):
    bm = 1000
    return pl.pallas_call(
        _fin_body,
        grid=(N // bm,),
        in_specs=[
            pl.BlockSpec((NC, bm, DH), lambda i: (0, i, 0)),
            pl.BlockSpec((1, D), lambda i: (0, 0)),
            pl.BlockSpec((1, D), lambda i: (0, 0)),
            pl.BlockSpec((1, D), lambda i: (0, 0)),
        ],
        out_specs=pl.BlockSpec((bm, D), lambda i: (i, 0)),
        out_shape=jax.ShapeDtypeStruct((N, D), jnp.float32),
    )(partials, b, gamma, beta)


_PERM = np.stack([np.arange(16), np.arange(16) + 16], 1).reshape(32)
_PERM = (_PERM[None, :] + 32 * np.arange(D // 32)[:, None]).reshape(D)


@jax.jit
def kernel(adj_indices, adj_values, features, W, b, gamma, beta):
    base = _tc_matmul(features, W)
    base_p = base.reshape(N, D // 32, 2, 16).transpose(0, 1, 3, 2)
    base_bf = base_p.reshape(N, D).astype(jnp.bfloat16)
    base32 = lax.bitcast_convert_type(
        base_bf.reshape(N, NC, WH, 2), jnp.int32)
    base32 = base32.transpose(1, 0, 2)
    pad = NS * EW - E
    rowcol = jnp.concatenate(
        [adj_indices[0] << 16 | adj_indices[1], jnp.zeros((pad,), jnp.int32)])
    val = jnp.concatenate([adj_values, jnp.zeros((pad,), jnp.float32)])
    partials = _sc_spmm(base32,
                        rowcol.reshape(NS, NCHUNK, CHUNK),
                        val.reshape(NS, NCHUNK, CHUNK))
    return _tc_finish(partials, b,
                      gamma.reshape(1, D), beta.reshape(1, D))
